# baseline (speedup 1.0000x reference)
import math

import jax
import jax.numpy as jnp
import numpy as np
from jax import lax
from jax.experimental import pallas as pl
from jax.experimental.pallas import tpu as pltpu
from jax.experimental.pallas import tpu_sc as plsc

D_MODEL = 128
SEQ = 200
BATCH = 4096
SCALE = math.sqrt(128.0)
LANES = 16
NW = 32
NBPW = BATCH // NW
NST = NBPW * 2
NBUF = 8
DIST = NBUF // 2
CH = (104, 96)


def _positional_encoding(length, depth):
    half = depth / 2
    positions = np.arange(length)[:, np.newaxis]
    depths = np.arange(half)[np.newaxis, :] / half
    angle_rates = 1 / 1000 ** depths
    angle_rads = positions * angle_rates
    return np.concatenate(
        [np.sin(angle_rads), np.cos(angle_rads)], axis=-1
    ).astype(np.float32)


_PE = _positional_encoding(SEQ, D_MODEL)


def _sc_body(x_ref, table_ref, pe_hbm, out_ref, pe_v, idx0, idx1, *scratch):
    rows = scratch[0:NBUF]
    g = scratch[NBUF:2 * NBUF]
    st = scratch[2 * NBUF:3 * NBUF]
    idx = (idx0, idx1)

    c = lax.axis_index("c")
    s = lax.axis_index("s")
    wid = s * 2 + c
    batch0 = wid * NBPW
    pltpu.sync_copy(pe_hbm, pe_v)

    def load_idx_block(q, blk):
        pltpu.sync_copy(x_ref.at[pl.ds((batch0 + 2 * q) * SEQ, 2 * SEQ)],
                        idx[blk])

    _POFF = (0, CH[0], SEQ, SEQ + CH[0])

    def start_gather(p, blk, j):
        pltpu.async_copy(
            table_ref.at[idx[blk].at[pl.ds(_POFF[p], CH[j % 2])]],
            rows[j], g[j])

    def wait_gather(j):
        pltpu.make_async_copy(
            table_ref.at[idx0.at[pl.ds(0, CH[j % 2])]], rows[j], g[j]).wait()

    def _out_slice(k, j):
        row0 = (batch0 + 4 * k + j // 2) * SEQ + (j % 2) * CH[0]
        return out_ref.at[pl.ds(row0, CH[j % 2])]

    def start_store(k, j):
        pltpu.async_copy(rows[j], _out_slice(k, j), st[j])

    def wait_store(k, j):
        pltpu.make_async_copy(rows[j], _out_slice(k, j), st[j]).wait()

    def compute_pair(ja, jb):
        ra, rb = rows[ja], rows[jb]
        pbase = (ja % 2) * CH[0]

        def row_body(r, carry):
            for v in range(D_MODEL // LANES):
                sl = pl.ds(v * LANES, LANES)
                pv = pe_v[pbase + r, sl]
                ra[r, sl] = ra[r, sl] * SCALE + pv
                rb[r, sl] = rb[r, sl] * SCALE + pv
            return carry

        lax.fori_loop(0, CH[ja % 2], row_body, 0)

    load_idx_block(0, 0)
    for j in range(DIST):
        start_gather(j, 0, j)

    K = NST // NBUF

    def iter_body(k, carry):
        for j in range(NBUF):
            if j == 0:
                load_idx_block(2 * k + 1, 1)
            if j == DIST:
                @pl.when(k < K - 1)
                def _ld():
                    load_idx_block(2 * k + 2, 0)

            if j in (2, 3):
                wait_gather(j - 2)
                wait_gather(j)
                compute_pair(j - 2, j)
                start_store(k, j - 2)
                start_store(k, j)
            elif j in (6, 7):
                wait_gather(j - 2)
                wait_gather(j)
                compute_pair(j - 2, j)
                start_store(k, j - 2)
                start_store(k, j)

            jn = (j + DIST) % NBUF
            if j < DIST:
                @pl.when(k > 0)
                def _w():
                    wait_store(k - 1, jn)

                start_gather(j, 1, jn)
            else:
                @pl.when(k < K - 1)
                def _w2():
                    wait_store(k, jn)
                    start_gather(j - DIST, 0, jn)
        return carry

    lax.fori_loop(0, K, iter_body, 0)

    for j in range(NBUF):
        wait_store(K - 1, j)


@jax.jit
def _impl(x, table):
    xf = x.reshape(-1)
    mesh = plsc.VectorSubcoreMesh(core_axis_name="c", subcore_axis_name="s")
    scratch = (
        [pltpu.VMEM((SEQ, D_MODEL), jnp.float32)]
        + [pltpu.VMEM((2 * SEQ,), jnp.int32)] * 2
        + [pltpu.VMEM((CH[j % 2], D_MODEL), jnp.float32)
           for j in range(NBUF)]
        + [pltpu.SemaphoreType.DMA] * NBUF
        + [pltpu.SemaphoreType.DMA] * NBUF
    )
    out = pl.kernel(
        _sc_body,
        out_type=jax.ShapeDtypeStruct((BATCH * SEQ, D_MODEL), jnp.float32),
        mesh=mesh,
        scratch_types=scratch,
    )(xf, table, jnp.asarray(_PE))
    return out.reshape(BATCH, SEQ, D_MODEL)


def kernel(x, table):
    return _impl(x, table)
